# baseline (device time: 208163 ns/iter reference)
import jax
import jax.numpy as jnp
from jax import lax
from jax.experimental import pallas as pl
from jax.experimental.pallas import tpu as pltpu

N_DEV = 16


def kernel(x, w_mat):
    m_per, k = x.shape
    _, n_per = w_mat.shape

    def body(x_ref, w_ref, out_ref, comm_ref, send_sems, recv_sems):
        my_pos = lax.axis_index("i")
        left = lax.rem(my_pos + N_DEV - 1, N_DEV)
        right = lax.rem(my_pos + 1, N_DEV)

        barrier_sem = pltpu.get_barrier_semaphore()
        for nbr in (left, right):
            pl.semaphore_signal(
                barrier_sem, inc=1,
                device_id=(nbr,), device_id_type=pl.DeviceIdType.MESH,
            )
        pl.semaphore_wait(barrier_sem, 2)

        comm_ref[0] = x_ref[...]
        out_ref[pl.ds(my_pos * m_per, m_per), :] = jnp.dot(
            x_ref[...], w_ref[...], preferred_element_type=jnp.float32
        )

        for h in range(N_DEV - 1):
            rdma = pltpu.make_async_remote_copy(
                src_ref=comm_ref.at[h],
                dst_ref=comm_ref.at[h + 1],
                send_sem=send_sems.at[h],
                recv_sem=recv_sems.at[h + 1],
                device_id=(right,),
                device_id_type=pl.DeviceIdType.MESH,
            )
            rdma.start()
            rdma.wait()
            origin = lax.rem(my_pos + N_DEV - h - 1, N_DEV)
            out_ref[pl.ds(origin * m_per, m_per), :] = jnp.dot(
                comm_ref[h + 1], w_ref[...], preferred_element_type=jnp.float32
            )

    return pl.pallas_call(
        body,
        out_shape=jax.ShapeDtypeStruct((N_DEV * m_per, n_per), jnp.float32),
        in_specs=[
            pl.BlockSpec(memory_space=pltpu.VMEM),
            pl.BlockSpec(memory_space=pltpu.VMEM),
        ],
        out_specs=pl.BlockSpec(memory_space=pltpu.VMEM),
        scratch_shapes=[
            pltpu.VMEM((N_DEV, m_per, k), jnp.float32),
            pltpu.SemaphoreType.DMA((N_DEV,)),
            pltpu.SemaphoreType.DMA((N_DEV,)),
        ],
        compiler_params=pltpu.CompilerParams(collective_id=0),
    )(x, w_mat)


# device time: 116698 ns/iter; 1.7838x vs baseline; 1.7838x over previous
import jax
import jax.numpy as jnp
from jax import lax
from jax.experimental import pallas as pl
from jax.experimental.pallas import tpu as pltpu

N_DEV = 16

RING = (0, 4, 8, 12, 15, 11, 7, 3, 2, 6, 10, 14, 13, 9, 5, 1)

N_CW = 8
N_CCW = 7


def _ring_shift(my_pos, offset):
    val = jnp.int32(0)
    for k in range(N_DEV):
        val += jnp.where(
            my_pos == RING[k], jnp.int32(RING[(k + offset) % N_DEV]), 0
        )
    return val


def kernel(x, w_mat):
    m_per, k = x.shape
    _, n_per = w_mat.shape

    def body(x_ref, w_ref, out_ref, cw_ref, ccw_ref,
             cw_send, cw_recv, ccw_send, ccw_recv):
        my_pos = lax.axis_index("i")
        right = _ring_shift(my_pos, 1)
        left = _ring_shift(my_pos, -1)

        barrier_sem = pltpu.get_barrier_semaphore()
        for nbr in (left, right):
            pl.semaphore_signal(
                barrier_sem, inc=1,
                device_id=(nbr,), device_id_type=pl.DeviceIdType.MESH,
            )
        pl.semaphore_wait(barrier_sem, 2)

        cw_ref[0] = x_ref[...]
        ccw_ref[0] = x_ref[...]
        out_ref[pl.ds(my_pos * m_per, m_per), :] = jnp.dot(
            x_ref[...], w_ref[...], preferred_element_type=jnp.float32
        )

        for s in range(N_CW):
            cw = pltpu.make_async_remote_copy(
                src_ref=cw_ref.at[s],
                dst_ref=cw_ref.at[s + 1],
                send_sem=cw_send.at[s],
                recv_sem=cw_recv.at[s + 1],
                device_id=(right,),
                device_id_type=pl.DeviceIdType.MESH,
            )
            cw.start()
            if s < N_CCW:
                ccw = pltpu.make_async_remote_copy(
                    src_ref=ccw_ref.at[s],
                    dst_ref=ccw_ref.at[s + 1],
                    send_sem=ccw_send.at[s],
                    recv_sem=ccw_recv.at[s + 1],
                    device_id=(left,),
                    device_id_type=pl.DeviceIdType.MESH,
                )
                ccw.start()
            cw.wait()
            origin = _ring_shift(my_pos, -(s + 1))
            out_ref[pl.ds(origin * m_per, m_per), :] = jnp.dot(
                cw_ref[s + 1], w_ref[...], preferred_element_type=jnp.float32
            )
            if s < N_CCW:
                ccw.wait()
                origin = _ring_shift(my_pos, s + 1)
                out_ref[pl.ds(origin * m_per, m_per), :] = jnp.dot(
                    ccw_ref[s + 1], w_ref[...],
                    preferred_element_type=jnp.float32,
                )

    return pl.pallas_call(
        body,
        out_shape=jax.ShapeDtypeStruct((N_DEV * m_per, n_per), jnp.float32),
        in_specs=[
            pl.BlockSpec(memory_space=pltpu.VMEM),
            pl.BlockSpec(memory_space=pltpu.VMEM),
        ],
        out_specs=pl.BlockSpec(memory_space=pltpu.VMEM),
        scratch_shapes=[
            pltpu.VMEM((N_CW + 1, m_per, k), jnp.float32),
            pltpu.VMEM((N_CCW + 1, m_per, k), jnp.float32),
            pltpu.SemaphoreType.DMA((N_CW,)),
            pltpu.SemaphoreType.DMA((N_CW + 1,)),
            pltpu.SemaphoreType.DMA((N_CCW,)),
            pltpu.SemaphoreType.DMA((N_CCW + 1,)),
        ],
        compiler_params=pltpu.CompilerParams(collective_id=0),
    )(x, w_mat)


# device time: 95690 ns/iter; 2.1754x vs baseline; 1.2195x over previous
import jax
import jax.numpy as jnp
from jax import lax
from jax.experimental import pallas as pl
from jax.experimental.pallas import tpu as pltpu

N_DEV = 16
N_HOP = 8

RING = (0, 4, 8, 12, 15, 11, 7, 3, 2, 6, 10, 14, 13, 9, 5, 1)


def _ring_shift(my_pos, offset):
    val = jnp.int32(0)
    for k in range(N_DEV):
        val += jnp.where(
            my_pos == RING[k], jnp.int32(RING[(k + offset) % N_DEV]), 0
        )
    return val


def kernel(x, w_mat):
    m_per, k = x.shape
    _, n_per = w_mat.shape
    m_half = m_per // 2

    def body(x_ref, w_ref, out_ref, cw_ref, ccw_ref,
             cw_send, cw_recv, ccw_send, ccw_recv):
        my_pos = lax.axis_index("i")
        right = _ring_shift(my_pos, 1)
        left = _ring_shift(my_pos, -1)

        barrier_sem = pltpu.get_barrier_semaphore()
        for nbr in (left, right):
            pl.semaphore_signal(
                barrier_sem, inc=1,
                device_id=(nbr,), device_id_type=pl.DeviceIdType.MESH,
            )
        pl.semaphore_wait(barrier_sem, 2)

        started = []

        def fwd(buf_ref, src_slot, sub, send_sems, recv_sems, dev):
            r = pltpu.make_async_remote_copy(
                src_ref=(x_ref.at[pl.ds(sub * m_half, m_half)]
                         if src_slot == 0
                         else buf_ref.at[src_slot, pl.ds(sub * m_half, m_half)]),
                dst_ref=buf_ref.at[src_slot + 1, pl.ds(sub * m_half, m_half)],
                send_sem=send_sems.at[src_slot, sub],
                recv_sem=recv_sems.at[src_slot + 1, sub],
                device_id=(dev,),
                device_id_type=pl.DeviceIdType.MESH,
            )
            r.start()
            started.append(r)

        def wait_in(buf_ref, slot, sub, recv_sems):
            pltpu.make_async_remote_copy(
                src_ref=x_ref.at[pl.ds(sub * m_half, m_half)],
                dst_ref=buf_ref.at[slot, pl.ds(sub * m_half, m_half)],
                send_sem=recv_sems.at[slot, sub],
                recv_sem=recv_sems.at[slot, sub],
                device_id=(left,),
                device_id_type=pl.DeviceIdType.MESH,
            ).wait_recv()

        for sub in (0, 1):
            fwd(cw_ref, 0, sub, cw_send, cw_recv, right)
            fwd(ccw_ref, 0, sub, ccw_send, ccw_recv, left)
        out_ref[pl.ds(my_pos * m_per, m_per), :] = jnp.dot(
            x_ref[...], w_ref[...], preferred_element_type=jnp.float32
        )

        for s in range(1, N_HOP + 1):
            if s <= 6:
                for sub in (0, 1):
                    wait_in(cw_ref, s, sub, cw_recv)
                    fwd(cw_ref, s, sub, cw_send, cw_recv, right)
                    wait_in(ccw_ref, s, sub, ccw_recv)
                    fwd(ccw_ref, s, sub, ccw_send, ccw_recv, left)
            elif s == 7:
                wait_in(cw_ref, 7, 0, cw_recv)
                fwd(cw_ref, 7, 0, cw_send, cw_recv, right)
                wait_in(ccw_ref, 7, 1, ccw_recv)
                fwd(ccw_ref, 7, 1, ccw_send, ccw_recv, left)
                wait_in(cw_ref, 7, 1, cw_recv)
                wait_in(ccw_ref, 7, 0, ccw_recv)
            else:
                wait_in(cw_ref, 8, 0, cw_recv)
                wait_in(ccw_ref, 8, 1, ccw_recv)

            if s <= 7:
                origin = _ring_shift(my_pos, -s)
                out_ref[pl.ds(origin * m_per, m_per), :] = jnp.dot(
                    cw_ref[s], w_ref[...], preferred_element_type=jnp.float32
                )
                origin = _ring_shift(my_pos, s)
                out_ref[pl.ds(origin * m_per, m_per), :] = jnp.dot(
                    ccw_ref[s], w_ref[...], preferred_element_type=jnp.float32
                )
            else:
                o8 = _ring_shift(my_pos, N_HOP)
                out_ref[pl.ds(o8 * m_per, m_half), :] = jnp.dot(
                    cw_ref[8, :m_half], w_ref[...],
                    preferred_element_type=jnp.float32,
                )
                out_ref[pl.ds(o8 * m_per + m_half, m_half), :] = jnp.dot(
                    ccw_ref[8, m_half:], w_ref[...],
                    preferred_element_type=jnp.float32,
                )

        for r in started:
            r.wait_send()

    return pl.pallas_call(
        body,
        out_shape=jax.ShapeDtypeStruct((N_DEV * m_per, n_per), jnp.float32),
        in_specs=[
            pl.BlockSpec(memory_space=pltpu.VMEM),
            pl.BlockSpec(memory_space=pltpu.VMEM),
        ],
        out_specs=pl.BlockSpec(memory_space=pltpu.VMEM),
        scratch_shapes=[
            pltpu.VMEM((N_HOP + 1, m_per, k), jnp.float32),
            pltpu.VMEM((N_HOP + 1, m_per, k), jnp.float32),
            pltpu.SemaphoreType.DMA((N_HOP, 2)),
            pltpu.SemaphoreType.DMA((N_HOP + 1, 2)),
            pltpu.SemaphoreType.DMA((N_HOP, 2)),
            pltpu.SemaphoreType.DMA((N_HOP + 1, 2)),
        ],
        compiler_params=pltpu.CompilerParams(collective_id=0),
    )(x, w_mat)


# device time: 94159 ns/iter; 2.2108x vs baseline; 1.0163x over previous
import jax
import jax.numpy as jnp
from jax import lax
from jax.experimental import pallas as pl
from jax.experimental.pallas import tpu as pltpu

N_DEV = 16
N_HOP = 8
N_SUB = 4

RING = (0, 4, 8, 12, 15, 11, 7, 3, 2, 6, 10, 14, 13, 9, 5, 1)


def _ring_shift(my_pos, offset):
    val = jnp.int32(0)
    for k in range(N_DEV):
        val += jnp.where(
            my_pos == RING[k], jnp.int32(RING[(k + offset) % N_DEV]), 0
        )
    return val


def kernel(x, w_mat):
    m_per, k = x.shape
    _, n_per = w_mat.shape
    m_sub = m_per // N_SUB
    half = N_SUB // 2

    cw_subs = tuple(range(N_SUB))
    ccw_subs = tuple(reversed(cw_subs))

    def body(x_ref, w_ref, out_ref, cw_ref, ccw_ref,
             cw_send, cw_recv, ccw_send, ccw_recv):
        my_pos = lax.axis_index("i")
        right = _ring_shift(my_pos, 1)
        left = _ring_shift(my_pos, -1)

        barrier_sem = pltpu.get_barrier_semaphore()
        for nbr in (left, right):
            pl.semaphore_signal(
                barrier_sem, inc=1,
                device_id=(nbr,), device_id_type=pl.DeviceIdType.MESH,
            )
        pl.semaphore_wait(barrier_sem, 2)

        started = []

        def fwd(buf_ref, src_slot, sub, send_sems, recv_sems, dev):
            r = pltpu.make_async_remote_copy(
                src_ref=(x_ref.at[pl.ds(sub * m_sub, m_sub)]
                         if src_slot == 0
                         else buf_ref.at[src_slot, pl.ds(sub * m_sub, m_sub)]),
                dst_ref=buf_ref.at[src_slot + 1, pl.ds(sub * m_sub, m_sub)],
                send_sem=send_sems.at[src_slot, sub],
                recv_sem=recv_sems.at[src_slot + 1, sub],
                device_id=(dev,),
                device_id_type=pl.DeviceIdType.MESH,
            )
            r.start()
            started.append(r)

        def wait_in(buf_ref, slot, sub, recv_sems):
            pltpu.make_async_remote_copy(
                src_ref=x_ref.at[pl.ds(sub * m_sub, m_sub)],
                dst_ref=buf_ref.at[slot, pl.ds(sub * m_sub, m_sub)],
                send_sem=recv_sems.at[slot, sub],
                recv_sem=recv_sems.at[slot, sub],
                device_id=(left,),
                device_id_type=pl.DeviceIdType.MESH,
            ).wait_recv()

        for i in range(N_SUB):
            fwd(cw_ref, 0, cw_subs[i], cw_send, cw_recv, right)
            fwd(ccw_ref, 0, ccw_subs[i], ccw_send, ccw_recv, left)
        out_ref[pl.ds(my_pos * m_per, m_per), :] = jnp.dot(
            x_ref[...], w_ref[...], preferred_element_type=jnp.float32
        )

        for s in range(1, N_HOP + 1):
            if s <= 6:
                for i in range(N_SUB):
                    wait_in(cw_ref, s, cw_subs[i], cw_recv)
                    fwd(cw_ref, s, cw_subs[i], cw_send, cw_recv, right)
                    wait_in(ccw_ref, s, ccw_subs[i], ccw_recv)
                    fwd(ccw_ref, s, ccw_subs[i], ccw_send, ccw_recv, left)
            elif s == 7:
                for sub in cw_subs[:half]:
                    wait_in(cw_ref, 7, sub, cw_recv)
                    fwd(cw_ref, 7, sub, cw_send, cw_recv, right)
                for sub in ccw_subs[:half]:
                    wait_in(ccw_ref, 7, sub, ccw_recv)
                    fwd(ccw_ref, 7, sub, ccw_send, ccw_recv, left)
                for sub in cw_subs[half:]:
                    wait_in(cw_ref, 7, sub, cw_recv)
                for sub in ccw_subs[half:]:
                    wait_in(ccw_ref, 7, sub, ccw_recv)
            else:
                for sub in cw_subs[:half]:
                    wait_in(cw_ref, 8, sub, cw_recv)
                for sub in ccw_subs[:half]:
                    wait_in(ccw_ref, 8, sub, ccw_recv)

            if s <= 7:
                origin = _ring_shift(my_pos, -s)
                out_ref[pl.ds(origin * m_per, m_per), :] = jnp.dot(
                    cw_ref[s], w_ref[...], preferred_element_type=jnp.float32
                )
                origin = _ring_shift(my_pos, s)
                out_ref[pl.ds(origin * m_per, m_per), :] = jnp.dot(
                    ccw_ref[s], w_ref[...], preferred_element_type=jnp.float32
                )
            else:
                o8 = _ring_shift(my_pos, N_HOP)
                m_half = m_per // 2
                out_ref[pl.ds(o8 * m_per, m_half), :] = jnp.dot(
                    cw_ref[8, :m_half], w_ref[...],
                    preferred_element_type=jnp.float32,
                )
                out_ref[pl.ds(o8 * m_per + m_half, m_half), :] = jnp.dot(
                    ccw_ref[8, m_half:], w_ref[...],
                    preferred_element_type=jnp.float32,
                )

        for r in started:
            r.wait_send()

    return pl.pallas_call(
        body,
        out_shape=jax.ShapeDtypeStruct((N_DEV * m_per, n_per), jnp.float32),
        in_specs=[
            pl.BlockSpec(memory_space=pltpu.VMEM),
            pl.BlockSpec(memory_space=pltpu.VMEM),
        ],
        out_specs=pl.BlockSpec(memory_space=pltpu.VMEM),
        scratch_shapes=[
            pltpu.VMEM((N_HOP + 1, m_per, k), jnp.float32),
            pltpu.VMEM((N_HOP + 1, m_per, k), jnp.float32),
            pltpu.SemaphoreType.DMA((N_HOP, N_SUB)),
            pltpu.SemaphoreType.DMA((N_HOP + 1, N_SUB)),
            pltpu.SemaphoreType.DMA((N_HOP, N_SUB)),
            pltpu.SemaphoreType.DMA((N_HOP + 1, N_SUB)),
        ],
        compiler_params=pltpu.CompilerParams(collective_id=0),
    )(x, w_mat)


# device time: 67333 ns/iter; 3.0915x vs baseline; 1.3984x over previous
import jax
import jax.numpy as jnp
from jax import lax
from jax.experimental import pallas as pl
from jax.experimental.pallas import tpu as pltpu

N_DEV = 16
D = 5
N_CHORD = 5
N_SUB = 2

RING = (0, 4, 8, 12, 15, 11, 7, 3, 2, 6, 10, 14, 13, 9, 5, 1)

PARTNER_OFF = (7, 5, -5, -7)
CHORD_ORIGIN = (
    (7, 6, 8, -7, -6),
    (6, 7, 8, -7, -6),
    (-6, -7, 8, 7, 6),
    (-7, -6, 8, 7, 6),
)
CHORD_FROM_RING = {
    (0, 1, 1): 1, (0, 0, 1): 2, (0, 0, 2): 3, (0, 0, 3): 4,
    (1, 0, 1): 0, (1, 0, 2): 1, (1, 0, 3): 2, (1, 0, 4): 3, (1, 0, 5): 4,
    (2, 1, 1): 0, (2, 1, 2): 1, (2, 1, 3): 2, (2, 1, 4): 3, (2, 1, 5): 4,
    (3, 0, 1): 1, (3, 1, 1): 2, (3, 1, 2): 3, (3, 1, 3): 4,
}


def _ring_index(my_pos):
    r = jnp.int32(0)
    for k in range(N_DEV):
        r += jnp.where(my_pos == RING[k], jnp.int32(k), 0)
    return r


def _ring_shift(my_pos, offset):
    val = jnp.int32(0)
    for k in range(N_DEV):
        val += jnp.where(
            my_pos == RING[k], jnp.int32(RING[(k + offset) % N_DEV]), 0
        )
    return val


def kernel(x, w_mat):
    m_per, k = x.shape
    _, n_per = w_mat.shape
    m_sub = m_per // N_SUB

    def body(x_ref, w_ref, out_ref, cw_ref, ccw_ref, ch_ref,
             cw_send, cw_recv, ccw_send, ccw_recv, ch_send, ch_recv):
        my_pos = lax.axis_index("i")
        r_idx = _ring_index(my_pos)
        flavor = lax.rem(r_idx, 4)
        right = _ring_shift(my_pos, 1)
        left = _ring_shift(my_pos, -1)
        partner = jnp.int32(0)
        for f in range(4):
            partner += jnp.where(
                flavor == f, _ring_shift(my_pos, PARTNER_OFF[f]), 0
            )

        barrier_sem = pltpu.get_barrier_semaphore()
        for nbr in (left, right, partner):
            pl.semaphore_signal(
                barrier_sem, inc=1,
                device_id=(nbr,), device_id_type=pl.DeviceIdType.MESH,
            )
        pl.semaphore_wait(barrier_sem, 3)

        started = []

        def ring_fwd(buf_ref, src_slot, sub, send_sems, recv_sems, dev):
            r = pltpu.make_async_remote_copy(
                src_ref=(x_ref.at[pl.ds(sub * m_sub, m_sub)]
                         if src_slot == 0
                         else buf_ref.at[src_slot, pl.ds(sub * m_sub, m_sub)]),
                dst_ref=buf_ref.at[src_slot + 1, pl.ds(sub * m_sub, m_sub)],
                send_sem=send_sems.at[src_slot, sub],
                recv_sem=recv_sems.at[src_slot + 1, sub],
                device_id=(dev,),
                device_id_type=pl.DeviceIdType.MESH,
            )
            r.start()
            started.append(r)

        def chord_send(src_ref_slice, j, sub):
            pltpu.make_async_remote_copy(
                src_ref=src_ref_slice,
                dst_ref=ch_ref.at[j, pl.ds(sub * m_sub, m_sub)],
                send_sem=ch_send.at[j, sub],
                recv_sem=ch_recv.at[j, sub],
                device_id=(partner,),
                device_id_type=pl.DeviceIdType.MESH,
            ).start()

        def wait_in(buf_ref, slot, sub, recv_sems):
            pltpu.make_async_remote_copy(
                src_ref=x_ref.at[pl.ds(sub * m_sub, m_sub)],
                dst_ref=buf_ref.at[slot, pl.ds(sub * m_sub, m_sub)],
                send_sem=recv_sems.at[slot, sub],
                recv_sem=recv_sems.at[slot, sub],
                device_id=(left,),
                device_id_type=pl.DeviceIdType.MESH,
            ).wait_recv()

        def chord_relay(direction, slot, src_slice, sub):
            for f in range(4):
                j = CHORD_FROM_RING.get((f, direction, slot))
                if j is not None:
                    @pl.when(flavor == f)
                    def _():
                        chord_send(src_slice, j, sub)

        for sub in range(N_SUB):
            ring_fwd(cw_ref, 0, sub, cw_send, cw_recv, right)
            ring_fwd(ccw_ref, 0, sub, ccw_send, ccw_recv, left)

            @pl.when(jnp.logical_or(flavor == 0, flavor == 3))
            def _():
                chord_send(x_ref.at[pl.ds(sub * m_sub, m_sub)], 0, sub)

        out_ref[pl.ds(my_pos * m_per, m_per), :] = jnp.dot(
            x_ref[...], w_ref[...], preferred_element_type=jnp.float32
        )

        for s in range(1, D + 1):
            for sub in range(N_SUB):
                wait_in(cw_ref, s, sub, cw_recv)
                if s < D:
                    ring_fwd(cw_ref, s, sub, cw_send, cw_recv, right)
                chord_relay(0, s, cw_ref.at[s, pl.ds(sub * m_sub, m_sub)], sub)

                wait_in(ccw_ref, s, sub, ccw_recv)
                if s < D:
                    ring_fwd(ccw_ref, s, sub, ccw_send, ccw_recv, left)
                chord_relay(1, s, ccw_ref.at[s, pl.ds(sub * m_sub, m_sub)], sub)

            origin = _ring_shift(my_pos, -s)
            out_ref[pl.ds(origin * m_per, m_per), :] = jnp.dot(
                cw_ref[s], w_ref[...], preferred_element_type=jnp.float32
            )
            origin = _ring_shift(my_pos, s)
            out_ref[pl.ds(origin * m_per, m_per), :] = jnp.dot(
                ccw_ref[s], w_ref[...], preferred_element_type=jnp.float32
            )

        for j in range(N_CHORD):
            for sub in range(N_SUB):
                wait_in(ch_ref, j, sub, ch_recv)
            origin = jnp.int32(0)
            for f in range(4):
                origin += jnp.where(
                    flavor == f, _ring_shift(my_pos, CHORD_ORIGIN[f][j]), 0
                )
            out_ref[pl.ds(origin * m_per, m_per), :] = jnp.dot(
                ch_ref[j], w_ref[...], preferred_element_type=jnp.float32
            )

        for r in started:
            r.wait_send()
        for j in range(N_CHORD):
            for sub in range(N_SUB):
                pltpu.make_async_remote_copy(
                    src_ref=x_ref.at[pl.ds(sub * m_sub, m_sub)],
                    dst_ref=ch_ref.at[j, pl.ds(sub * m_sub, m_sub)],
                    send_sem=ch_send.at[j, sub],
                    recv_sem=ch_recv.at[j, sub],
                    device_id=(partner,),
                    device_id_type=pl.DeviceIdType.MESH,
                ).wait_send()

    return pl.pallas_call(
        body,
        out_shape=jax.ShapeDtypeStruct((N_DEV * m_per, n_per), jnp.float32),
        in_specs=[
            pl.BlockSpec(memory_space=pltpu.VMEM),
            pl.BlockSpec(memory_space=pltpu.VMEM),
        ],
        out_specs=pl.BlockSpec(memory_space=pltpu.VMEM),
        scratch_shapes=[
            pltpu.VMEM((D + 1, m_per, k), jnp.float32),
            pltpu.VMEM((D + 1, m_per, k), jnp.float32),
            pltpu.VMEM((N_CHORD, m_per, k), jnp.float32),
            pltpu.SemaphoreType.DMA((D, N_SUB)),
            pltpu.SemaphoreType.DMA((D + 1, N_SUB)),
            pltpu.SemaphoreType.DMA((D, N_SUB)),
            pltpu.SemaphoreType.DMA((D + 1, N_SUB)),
            pltpu.SemaphoreType.DMA((N_CHORD, N_SUB)),
            pltpu.SemaphoreType.DMA((N_CHORD, N_SUB)),
        ],
        compiler_params=pltpu.CompilerParams(collective_id=0),
    )(x, w_mat)
